# Initial kernel scaffold; baseline (speedup 1.0000x reference)
import jax
import jax.numpy as jnp
from jax.experimental import pallas as pl


def kernel(x, t_vertex, neighbor_index, v, adjweight, W, b):
    raise NotImplementedError("write your pallas kernel here")


# speedup vs baseline: 1.0138x; 1.0138x over previous
import functools

import jax
import jax.numpy as jnp
from jax import lax
from jax.experimental import pallas as pl
from jax.experimental.pallas import tpu as pltpu
from jax.experimental.pallas import tpu_sc as plsc

N_PTS = 10000
K_NBR = 16
FEATS = 128
S_DIM = 8

NW = 32
ROWS = N_PTS * K_NBR
B_PER_W = ROWS // NW
CHUNK = 40
NCHUNK = B_PER_W // CHUNK

NB = 400
GRID = N_PTS // NB


def _sc_gather(x2d, idx3):
    mesh = plsc.VectorSubcoreMesh(core_axis_name="c", subcore_axis_name="s")

    @functools.partial(
        pl.kernel,
        out_type=jax.ShapeDtypeStruct((ROWS, FEATS), jnp.float32),
        mesh=mesh,
        scratch_types=[
            pltpu.VMEM((NCHUNK, CHUNK), jnp.int32),
            pltpu.VMEM((CHUNK, FEATS), jnp.float32),
            pltpu.VMEM((CHUNK, FEATS), jnp.float32),
            pltpu.SemaphoreType.DMA,
            pltpu.SemaphoreType.DMA,
        ],
    )
    def k(x_hbm, idx_hbm, out_hbm, idx_v, buf0, buf1, sem0, sem1):
        wid = lax.axis_index("s") * 2 + lax.axis_index("c")
        base = wid * B_PER_W
        pltpu.sync_copy(idx_hbm.at[wid], idx_v)

        def step(j, carry):
            c0 = 2 * j
            c1 = c0 + 1
            g0 = pltpu.async_copy(x_hbm.at[idx_v.at[c0]], buf0, sem0)
            g1 = pltpu.async_copy(x_hbm.at[idx_v.at[c1]], buf1, sem1)
            g0.wait()
            pltpu.sync_copy(buf0, out_hbm.at[pl.ds(base + c0 * CHUNK, CHUNK)])
            g1.wait()
            pltpu.sync_copy(buf1, out_hbm.at[pl.ds(base + c1 * CHUNK, CHUNK)])
            return carry

        lax.fori_loop(0, NCHUNK // 2, step, 0)
        if NCHUNK % 2:
            c = NCHUNK - 1
            pltpu.async_copy(x_hbm.at[idx_v.at[c]], buf0, sem0).wait()
            pltpu.sync_copy(buf0, out_hbm.at[pl.ds(base + c * CHUNK, CHUNK)])

    return k(x2d, idx3)


def _tc_body(g_ref, v_ref, aw_ref, wt_ref, b_ref, o_ref):
    i = pl.program_id(0)
    adjw = jnp.dot(v_ref[...], aw_ref[...], preferred_element_type=jnp.float32)
    acc = None
    for t in range(K_NBR):
        xt = None
        for k in range(K_NBR):
            c = t * K_NBR + k
            term = adjw[:, c:c + 1] * g_ref[:, k, :]
            xt = term if xt is None else xt + term
        xt = jnp.where(xt > 0, xt, jnp.exp(xt) - 1.0)
        p = jnp.dot(xt, wt_ref[t * FEATS:(t + 1) * FEATS, :],
                    preferred_element_type=jnp.float32)
        acc = p if acc is None else acc + p
    y = acc + b_ref[...]
    y = jnp.where(y > 0, y, jnp.exp(y) - 1.0)
    rows = i * NB + lax.broadcasted_iota(jnp.int32, (NB, FEATS), 0)
    o_ref[...] = jnp.where(rows == N_PTS - 1, 0.0, y)


def _tc_compute(g3, v, awr, wt, b2, interpret=False):
    return pl.pallas_call(
        _tc_body,
        grid=(GRID,),
        in_specs=[
            pl.BlockSpec((NB, K_NBR, FEATS), lambda i: (i, 0, 0)),
            pl.BlockSpec((NB, S_DIM), lambda i: (i, 0)),
            pl.BlockSpec((S_DIM, K_NBR * K_NBR), lambda i: (0, 0)),
            pl.BlockSpec((K_NBR * FEATS, FEATS), lambda i: (0, 0)),
            pl.BlockSpec((1, FEATS), lambda i: (0, 0)),
        ],
        out_specs=pl.BlockSpec((NB, FEATS), lambda i: (i, 0)),
        out_shape=jax.ShapeDtypeStruct((N_PTS, FEATS), jnp.float32),
        interpret=interpret,
    )(g3, v, awr, wt, b2)


def kernel(x, t_vertex, neighbor_index, v, adjweight, W, b):
    x2d = x.reshape(N_PTS, FEATS)
    idx3 = neighbor_index.reshape(NW, NCHUNK, CHUNK)
    gathered = _sc_gather(x2d, idx3)
    g3 = gathered.reshape(N_PTS, K_NBR, FEATS)
    awr = adjweight.transpose(0, 2, 1).reshape(S_DIM, K_NBR * K_NBR)
    wt = W.T
    b2 = b.reshape(1, FEATS)
    out2 = _tc_compute(g3, v, awr, wt, b2)
    return out2.reshape(1, N_PTS, FEATS)


# speedup vs baseline: 1.5039x; 1.4835x over previous
import functools

import jax
import jax.numpy as jnp
from jax import lax
from jax.experimental import pallas as pl
from jax.experimental.pallas import tpu as pltpu
from jax.experimental.pallas import tpu_sc as plsc

N_PTS = 10000
K_NBR = 16
FEATS = 128
S_DIM = 8

NW = 32
ROWS = N_PTS * K_NBR
B_PER_W = ROWS // NW
CHUNK = 40
NCHUNK = B_PER_W // CHUNK

NB = 400
GRID = N_PTS // NB


def _sc_gather(x2d, idx3):
    mesh = plsc.VectorSubcoreMesh(core_axis_name="c", subcore_axis_name="s")

    @functools.partial(
        pl.kernel,
        out_type=jax.ShapeDtypeStruct((ROWS, FEATS), jnp.float32),
        mesh=mesh,
        scratch_types=[
            pltpu.VMEM((NCHUNK, CHUNK), jnp.int32),
            pltpu.VMEM((CHUNK, FEATS), jnp.float32),
            pltpu.VMEM((CHUNK, FEATS), jnp.float32),
            pltpu.SemaphoreType.DMA,
            pltpu.SemaphoreType.DMA,
        ],
    )
    def k(x_hbm, idx_hbm, out_hbm, idx_v, buf0, buf1, sem0, sem1):
        wid = lax.axis_index("s") * 2 + lax.axis_index("c")
        base = wid * B_PER_W
        pltpu.sync_copy(idx_hbm.at[wid], idx_v)

        def step(j, carry):
            c0 = 2 * j
            c1 = c0 + 1
            g0 = pltpu.async_copy(x_hbm.at[idx_v.at[c0]], buf0, sem0)
            g1 = pltpu.async_copy(x_hbm.at[idx_v.at[c1]], buf1, sem1)
            g0.wait()
            pltpu.sync_copy(buf0, out_hbm.at[pl.ds(base + c0 * CHUNK, CHUNK)])
            g1.wait()
            pltpu.sync_copy(buf1, out_hbm.at[pl.ds(base + c1 * CHUNK, CHUNK)])
            return carry

        lax.fori_loop(0, NCHUNK // 2, step, 0)
        if NCHUNK % 2:
            c = NCHUNK - 1
            pltpu.async_copy(x_hbm.at[idx_v.at[c]], buf0, sem0).wait()
            pltpu.sync_copy(buf0, out_hbm.at[pl.ds(base + c * CHUNK, CHUNK)])

    return k(x2d, idx3)


SUB = 400


def _tc_body(g_ref, v_ref, awr_ref, awb_ref, wt_ref, b_ref, o_ref):
    i = pl.program_id(0)
    for sub in range(NB // SUB):
        n0 = sub * SUB
        vs = v_ref[n0:n0 + SUB]
        adjw = jnp.dot(vs, awr_ref[...], preferred_element_type=jnp.float32)
        acc = None
        for t in range(K_NBR):
            xt = None
            for k in range(K_NBR):
                c = t * K_NBR + k
                gk = g_ref[k, n0:n0 + SUB, :]
                if k % 2 == 0:
                    wb = jnp.dot(vs, awb_ref[:, c * FEATS:(c + 1) * FEATS],
                                 preferred_element_type=jnp.float32)
                    term = wb * gk
                else:
                    term = adjw[:, c:c + 1] * gk
                xt = term if xt is None else xt + term
            xt = jnp.where(xt > 0, xt, jnp.exp(xt) - 1.0)
            p = jnp.dot(xt, wt_ref[t * FEATS:(t + 1) * FEATS, :],
                        preferred_element_type=jnp.float32)
            acc = p if acc is None else acc + p
        y = acc + b_ref[...]
        y = jnp.where(y > 0, y, jnp.exp(y) - 1.0)
        rows = i * NB + n0 + lax.broadcasted_iota(jnp.int32, (SUB, FEATS), 0)
        o_ref[n0:n0 + SUB, :] = jnp.where(rows == N_PTS - 1, 0.0, y)


def _tc_compute(g3, v, awr, awb, wt, b2, interpret=False):
    return pl.pallas_call(
        _tc_body,
        grid=(GRID,),
        in_specs=[
            pl.BlockSpec((K_NBR, NB, FEATS), lambda i: (0, i, 0)),
            pl.BlockSpec((NB, S_DIM), lambda i: (i, 0)),
            pl.BlockSpec((S_DIM, K_NBR * K_NBR), lambda i: (0, 0)),
            pl.BlockSpec((S_DIM, K_NBR * K_NBR * FEATS), lambda i: (0, 0)),
            pl.BlockSpec((K_NBR * FEATS, FEATS), lambda i: (0, 0)),
            pl.BlockSpec((1, FEATS), lambda i: (0, 0)),
        ],
        out_specs=pl.BlockSpec((NB, FEATS), lambda i: (i, 0)),
        out_shape=jax.ShapeDtypeStruct((N_PTS, FEATS), jnp.float32),
        interpret=interpret,
    )(g3, v, awr, awb, wt, b2)


def kernel(x, t_vertex, neighbor_index, v, adjweight, W, b):
    x2d = x.reshape(N_PTS, FEATS)
    idx3 = neighbor_index.reshape(N_PTS, K_NBR).T.reshape(NW, NCHUNK, CHUNK)
    gathered = _sc_gather(x2d, idx3)
    g3 = gathered.reshape(K_NBR, N_PTS, FEATS)
    awr = adjweight.transpose(0, 2, 1).reshape(S_DIM, K_NBR * K_NBR)
    awb = jnp.broadcast_to(awr[:, :, None],
                           (S_DIM, K_NBR * K_NBR, FEATS)).reshape(S_DIM, -1)
    wt = W.T
    b2 = b.reshape(1, FEATS)
    out2 = _tc_compute(g3, v, awr, awb, wt, b2)
    return out2.reshape(1, N_PTS, FEATS)


# speedup vs baseline: 1.6668x; 1.1084x over previous
import functools

import jax
import jax.numpy as jnp
from jax import lax
from jax.experimental import pallas as pl
from jax.experimental.pallas import tpu as pltpu
from jax.experimental.pallas import tpu_sc as plsc

N_PTS = 10000
K_NBR = 16
FEATS = 128
S_DIM = 8

NW = 32
ROWS = N_PTS * K_NBR
B_PER_W = ROWS // NW
CHUNK = 40
NCHUNK = B_PER_W // CHUNK

NB = 400
GRID = N_PTS // NB


def _sc_gather(x2d, idx3):
    mesh = plsc.VectorSubcoreMesh(core_axis_name="c", subcore_axis_name="s")

    @functools.partial(
        pl.kernel,
        out_type=jax.ShapeDtypeStruct((ROWS, FEATS), jnp.float32),
        mesh=mesh,
        scratch_types=[
            pltpu.VMEM((NCHUNK, CHUNK), jnp.int32),
            pltpu.VMEM((CHUNK, FEATS), jnp.float32),
            pltpu.VMEM((CHUNK, FEATS), jnp.float32),
            pltpu.VMEM((CHUNK, FEATS), jnp.float32),
            pltpu.VMEM((CHUNK, FEATS), jnp.float32),
            pltpu.SemaphoreType.DMA,
            pltpu.SemaphoreType.DMA,
            pltpu.SemaphoreType.DMA,
            pltpu.SemaphoreType.DMA,
        ],
    )
    def k(x_hbm, idx_hbm, out_hbm, idx_v, b0, b1, b2, b3, s0, s1, s2, s3):
        wid = lax.axis_index("s") * 2 + lax.axis_index("c")
        base = wid * B_PER_W
        bufs = (b0, b1, b2, b3)
        sems = (s0, s1, s2, s3)
        pltpu.sync_copy(idx_hbm.at[wid], idx_v)

        def gather(c, l):
            pltpu.make_async_copy(x_hbm.at[idx_v.at[c]], bufs[l], sems[l]).start()

        def gather_wait(l):
            pltpu.make_async_copy(x_hbm.at[pl.ds(0, CHUNK)], bufs[l], sems[l]).wait()

        def write(c, l):
            pltpu.make_async_copy(bufs[l], out_hbm.at[pl.ds(base + c * CHUNK, CHUNK)],
                                  sems[l]).start()

        def write_wait(l):
            pltpu.make_async_copy(bufs[l], out_hbm.at[pl.ds(base, CHUNK)],
                                  sems[l]).wait()

        for l in range(4):
            gather(l, l)

        def step(j, carry):
            for l in range(4):
                gather_wait(l)
                write(4 * j + l, l)
            for l in range(4):
                write_wait(l)
                gather(4 * j + 4 + l, l)
            return carry

        lax.fori_loop(0, NCHUNK // 4 - 1, step, 0)
        for l in range(4):
            gather_wait(l)
            write(120 + l, l)
        write_wait(0)
        gather(NCHUNK - 1, 0)
        gather_wait(0)
        write(NCHUNK - 1, 0)
        write_wait(0)
        for l in range(1, 4):
            write_wait(l)

    return k(x2d, idx3)


SUB = 400


def _tc_body(g_ref, v_ref, awr_ref, awb_ref, wt_ref, b_ref, o_ref):
    i = pl.program_id(0)
    for sub in range(NB // SUB):
        n0 = sub * SUB
        vs = v_ref[n0:n0 + SUB]
        adjw = jnp.dot(vs, awr_ref[...], preferred_element_type=jnp.float32)
        acc = None
        for t in range(K_NBR):
            xt = None
            for k in range(K_NBR):
                c = t * K_NBR + k
                gk = g_ref[k, n0:n0 + SUB, :]
                if k % 2 == 0:
                    wb = jnp.dot(vs, awb_ref[:, c * FEATS:(c + 1) * FEATS],
                                 preferred_element_type=jnp.float32)
                    term = wb * gk
                else:
                    term = adjw[:, c:c + 1] * gk
                xt = term if xt is None else xt + term
            xt = jnp.where(xt > 0, xt, jnp.exp(xt) - 1.0)
            p = jnp.dot(xt, wt_ref[t * FEATS:(t + 1) * FEATS, :],
                        preferred_element_type=jnp.float32)
            acc = p if acc is None else acc + p
        y = acc + b_ref[...]
        y = jnp.where(y > 0, y, jnp.exp(y) - 1.0)
        rows = i * NB + n0 + lax.broadcasted_iota(jnp.int32, (SUB, FEATS), 0)
        o_ref[n0:n0 + SUB, :] = jnp.where(rows == N_PTS - 1, 0.0, y)


def _tc_compute(g3, v, awr, awb, wt, b2, interpret=False):
    return pl.pallas_call(
        _tc_body,
        grid=(GRID,),
        in_specs=[
            pl.BlockSpec((K_NBR, NB, FEATS), lambda i: (0, i, 0)),
            pl.BlockSpec((NB, S_DIM), lambda i: (i, 0)),
            pl.BlockSpec((S_DIM, K_NBR * K_NBR), lambda i: (0, 0)),
            pl.BlockSpec((S_DIM, K_NBR * K_NBR * FEATS), lambda i: (0, 0)),
            pl.BlockSpec((K_NBR * FEATS, FEATS), lambda i: (0, 0)),
            pl.BlockSpec((1, FEATS), lambda i: (0, 0)),
        ],
        out_specs=pl.BlockSpec((NB, FEATS), lambda i: (i, 0)),
        out_shape=jax.ShapeDtypeStruct((N_PTS, FEATS), jnp.float32),
        interpret=interpret,
    )(g3, v, awr, awb, wt, b2)


def kernel(x, t_vertex, neighbor_index, v, adjweight, W, b):
    x2d = x.reshape(N_PTS, FEATS)
    idx3 = neighbor_index.reshape(N_PTS, K_NBR).T.reshape(NW, NCHUNK, CHUNK)
    gathered = _sc_gather(x2d, idx3)
    g3 = gathered.reshape(K_NBR, N_PTS, FEATS)
    awr = adjweight.transpose(0, 2, 1).reshape(S_DIM, K_NBR * K_NBR)
    awb = jnp.broadcast_to(awr[:, :, None],
                           (S_DIM, K_NBR * K_NBR, FEATS)).reshape(S_DIM, -1)
    wt = W.T
    b2 = b.reshape(1, FEATS)
    out2 = _tc_compute(g3, v, awr, awb, wt, b2)
    return out2.reshape(1, N_PTS, FEATS)


# speedup vs baseline: 1.7706x; 1.0622x over previous
import functools

import jax
import jax.numpy as jnp
from jax import lax
from jax.experimental import pallas as pl
from jax.experimental.pallas import tpu as pltpu
from jax.experimental.pallas import tpu_sc as plsc

N_PTS = 10000
K_NBR = 16
FEATS = 128
S_DIM = 8

NW = 32
CHUNK = 40
NB = 400
P_SPLIT = 4000


def _sc_gather(x2d, idx3, n_pts):
    rows = n_pts * K_NBR
    b_per_w = rows // NW
    nchunk = b_per_w // CHUNK
    m = nchunk // 4
    r = nchunk - 4 * m
    mesh = plsc.VectorSubcoreMesh(core_axis_name="c", subcore_axis_name="s")

    @functools.partial(
        pl.kernel,
        out_type=jax.ShapeDtypeStruct((rows, FEATS), jnp.float32),
        mesh=mesh,
        scratch_types=[
            pltpu.VMEM((nchunk, CHUNK), jnp.int32),
            pltpu.VMEM((CHUNK, FEATS), jnp.float32),
            pltpu.VMEM((CHUNK, FEATS), jnp.float32),
            pltpu.VMEM((CHUNK, FEATS), jnp.float32),
            pltpu.VMEM((CHUNK, FEATS), jnp.float32),
            pltpu.SemaphoreType.DMA,
            pltpu.SemaphoreType.DMA,
            pltpu.SemaphoreType.DMA,
            pltpu.SemaphoreType.DMA,
        ],
    )
    def k(x_hbm, idx_hbm, out_hbm, idx_v, b0, b1, b2, b3, s0, s1, s2, s3):
        wid = lax.axis_index("s") * 2 + lax.axis_index("c")
        base = wid * b_per_w
        bufs = (b0, b1, b2, b3)
        sems = (s0, s1, s2, s3)
        pltpu.sync_copy(idx_hbm.at[wid], idx_v)

        def gather(c, l):
            pltpu.make_async_copy(x_hbm.at[idx_v.at[c]], bufs[l], sems[l]).start()

        def gather_wait(l):
            pltpu.make_async_copy(x_hbm.at[pl.ds(0, CHUNK)], bufs[l], sems[l]).wait()

        def write(c, l):
            pltpu.make_async_copy(bufs[l], out_hbm.at[pl.ds(base + c * CHUNK, CHUNK)],
                                  sems[l]).start()

        def write_wait(l):
            pltpu.make_async_copy(bufs[l], out_hbm.at[pl.ds(base, CHUNK)],
                                  sems[l]).wait()

        for l in range(4):
            gather(l, l)

        def step(j, carry):
            for l in range(4):
                gather_wait(l)
                write(4 * j + l, l)
            for l in range(4):
                write_wait(l)
                gather(4 * j + 4 + l, l)
            return carry

        lax.fori_loop(0, m - 1, step, 0)
        for l in range(4):
            gather_wait(l)
            write(4 * (m - 1) + l, l)
        for i in range(r):
            write_wait(i)
            gather(4 * m + i, i)
        for i in range(r):
            gather_wait(i)
            write(4 * m + i, i)
        for l in range(4):
            write_wait(l)

    return k(x2d, idx3)


def _tc_body(g_ref, v_ref, awr_ref, awb_ref, wt_ref, b_ref, o_ref, *, n_off):
    i = pl.program_id(0)
    vs = v_ref[...]
    adjw = jnp.dot(vs, awr_ref[...], preferred_element_type=jnp.float32)
    acc = None
    for t in range(K_NBR):
        xt = None
        for k in range(K_NBR):
            c = t * K_NBR + k
            gk = g_ref[k]
            if k % 2 == 0:
                wb = jnp.dot(vs, awb_ref[:, c * FEATS:(c + 1) * FEATS],
                             preferred_element_type=jnp.float32)
                term = wb * gk
            else:
                term = adjw[:, c:c + 1] * gk
            xt = term if xt is None else xt + term
        xt = jnp.where(xt > 0, xt, jnp.exp(xt) - 1.0)
        p = jnp.dot(xt, wt_ref[t * FEATS:(t + 1) * FEATS, :],
                    preferred_element_type=jnp.float32)
        acc = p if acc is None else acc + p
    y = acc + b_ref[...]
    y = jnp.where(y > 0, y, jnp.exp(y) - 1.0)
    rows = n_off + i * NB + lax.broadcasted_iota(jnp.int32, (NB, FEATS), 0)
    o_ref[...] = jnp.where(rows == N_PTS - 1, 0.0, y)


def _tc_compute(g3, v, awr, awb, wt, b2, n_off, interpret=False):
    n_pts = v.shape[0]
    return pl.pallas_call(
        functools.partial(_tc_body, n_off=n_off),
        grid=(n_pts // NB,),
        in_specs=[
            pl.BlockSpec((K_NBR, NB, FEATS), lambda i: (0, i, 0)),
            pl.BlockSpec((NB, S_DIM), lambda i: (i, 0)),
            pl.BlockSpec((S_DIM, K_NBR * K_NBR), lambda i: (0, 0)),
            pl.BlockSpec((S_DIM, K_NBR * K_NBR * FEATS), lambda i: (0, 0)),
            pl.BlockSpec((K_NBR * FEATS, FEATS), lambda i: (0, 0)),
            pl.BlockSpec((1, FEATS), lambda i: (0, 0)),
        ],
        out_specs=pl.BlockSpec((NB, FEATS), lambda i: (i, 0)),
        out_shape=jax.ShapeDtypeStruct((n_pts, FEATS), jnp.float32),
        interpret=interpret,
    )(g3, v, awr, awb, wt, b2)


def kernel(x, t_vertex, neighbor_index, v, adjweight, W, b):
    x2d = x.reshape(N_PTS, FEATS)
    idx_t = neighbor_index.reshape(N_PTS, K_NBR).T
    awr = adjweight.transpose(0, 2, 1).reshape(S_DIM, K_NBR * K_NBR)
    awb = jnp.broadcast_to(awr[:, :, None],
                           (S_DIM, K_NBR * K_NBR, FEATS)).reshape(S_DIM, -1)
    wt = W.T
    b2 = b.reshape(1, FEATS)

    outs = []
    for n0, n1 in ((0, P_SPLIT), (P_SPLIT, N_PTS)):
        n_pts = n1 - n0
        nchunk = n_pts * K_NBR // (NW * CHUNK)
        idx3 = idx_t[:, n0:n1].reshape(NW, nchunk, CHUNK)
        g3 = _sc_gather(x2d, idx3, n_pts).reshape(K_NBR, n_pts, FEATS)
        outs.append(_tc_compute(g3, v[n0:n1], awr, awb, wt, b2, n0))
    return jnp.concatenate(outs, axis=0).reshape(1, N_PTS, FEATS)


# speedup vs baseline: 1.8188x; 1.0272x over previous
import functools

import jax
import jax.numpy as jnp
from jax import lax
from jax.experimental import pallas as pl
from jax.experimental.pallas import tpu as pltpu
from jax.experimental.pallas import tpu_sc as plsc

N_PTS = 10000
K_NBR = 16
FEATS = 128
S_DIM = 8

NW = 32
CHUNK = 40
NB = 400
P_SPLIT = 4000


def _sc_gather(x2d, idx3, n_pts):
    rows = n_pts * K_NBR
    b_per_w = rows // NW
    nchunk = b_per_w // CHUNK
    m = nchunk // 4
    r = nchunk - 4 * m
    mesh = plsc.VectorSubcoreMesh(core_axis_name="c", subcore_axis_name="s")

    @functools.partial(
        pl.kernel,
        out_type=jax.ShapeDtypeStruct((rows, FEATS), jnp.float32),
        mesh=mesh,
        scratch_types=[
            pltpu.VMEM((nchunk, CHUNK), jnp.int32),
            pltpu.VMEM((CHUNK, FEATS), jnp.float32),
            pltpu.VMEM((CHUNK, FEATS), jnp.float32),
            pltpu.VMEM((CHUNK, FEATS), jnp.float32),
            pltpu.VMEM((CHUNK, FEATS), jnp.float32),
            pltpu.SemaphoreType.DMA,
            pltpu.SemaphoreType.DMA,
            pltpu.SemaphoreType.DMA,
            pltpu.SemaphoreType.DMA,
        ],
    )
    def k(x_hbm, idx_hbm, out_hbm, idx_v, b0, b1, b2, b3, s0, s1, s2, s3):
        wid = lax.axis_index("s") * 2 + lax.axis_index("c")
        base = wid * b_per_w
        bufs = (b0, b1, b2, b3)
        sems = (s0, s1, s2, s3)
        pltpu.sync_copy(idx_hbm.at[wid], idx_v)

        def gather(c, l):
            pltpu.make_async_copy(x_hbm.at[idx_v.at[c]], bufs[l], sems[l]).start()

        def gather_wait(l):
            pltpu.make_async_copy(x_hbm.at[pl.ds(0, CHUNK)], bufs[l], sems[l]).wait()

        def write(c, l):
            pltpu.make_async_copy(bufs[l], out_hbm.at[pl.ds(base + c * CHUNK, CHUNK)],
                                  sems[l]).start()

        def write_wait(l):
            pltpu.make_async_copy(bufs[l], out_hbm.at[pl.ds(base, CHUNK)],
                                  sems[l]).wait()

        for l in range(4):
            gather(l, l)

        def step(j, carry):
            for l in range(4):
                gather_wait(l)
                write(4 * j + l, l)
            for l in range(4):
                write_wait(l)
                gather(4 * j + 4 + l, l)
            return carry

        lax.fori_loop(0, m - 1, step, 0)
        for l in range(4):
            gather_wait(l)
            write(4 * (m - 1) + l, l)
        for i in range(r):
            write_wait(i)
            gather(4 * m + i, i)
        for i in range(r):
            gather_wait(i)
            write(4 * m + i, i)
        for l in range(4):
            write_wait(l)

    return k(x2d, idx3)


def _tc_body(g_ref, v_ref, awr_ref, awb_ref, wt_ref, b_ref, o_ref, *, n_off):
    i = pl.program_id(0)
    vs = v_ref[...]
    adjw = jnp.dot(vs, awr_ref[...], preferred_element_type=jnp.float32)
    acc = None
    for t in range(K_NBR):
        xt = None
        for k in range(K_NBR):
            c = t * K_NBR + k
            gk = g_ref[k]
            if k % 2 == 0:
                wb = jnp.dot(vs, awb_ref[:, c * FEATS:(c + 1) * FEATS],
                             preferred_element_type=jnp.float32)
                term = wb * gk
            else:
                term = adjw[:, c:c + 1] * gk
            xt = term if xt is None else xt + term
        xt = jnp.where(xt > 0, xt, jnp.exp(xt) - 1.0)
        p = jnp.dot(xt, wt_ref[t * FEATS:(t + 1) * FEATS, :],
                    preferred_element_type=jnp.float32)
        acc = p if acc is None else acc + p
    y = acc + b_ref[...]
    y = jnp.where(y > 0, y, jnp.exp(y) - 1.0)
    rows = n_off + i * NB + lax.broadcasted_iota(jnp.int32, (NB, FEATS), 0)
    o_ref[...] = jnp.where(rows == N_PTS - 1, 0.0, y)


def _tc_compute(g3, v, awr, awb, wt, b2, n_off, interpret=False):
    n_pts = v.shape[0]
    return pl.pallas_call(
        functools.partial(_tc_body, n_off=n_off),
        grid=(n_pts // NB,),
        in_specs=[
            pl.BlockSpec((K_NBR, NB, FEATS), lambda i: (0, i, 0)),
            pl.BlockSpec((NB, S_DIM), lambda i: (i, 0)),
            pl.BlockSpec((S_DIM, K_NBR * K_NBR), lambda i: (0, 0)),
            pl.BlockSpec((S_DIM, K_NBR * K_NBR * FEATS), lambda i: (0, 0)),
            pl.BlockSpec((K_NBR * FEATS, FEATS), lambda i: (0, 0)),
            pl.BlockSpec((1, FEATS), lambda i: (0, 0)),
        ],
        out_specs=pl.BlockSpec((NB, FEATS), lambda i: (i, 0)),
        out_shape=jax.ShapeDtypeStruct((n_pts, FEATS), jnp.float32),
        interpret=interpret,
    )(g3, v, awr, awb, wt, b2)


def kernel(x, t_vertex, neighbor_index, v, adjweight, W, b):
    x2d = x.reshape(N_PTS, FEATS)
    idx_t = neighbor_index.reshape(N_PTS, K_NBR).T
    awr = adjweight.transpose(0, 2, 1).reshape(S_DIM, K_NBR * K_NBR)
    awb = jnp.broadcast_to(awr[:, :, None],
                           (S_DIM, K_NBR * K_NBR, FEATS)).reshape(S_DIM, -1)
    wt = W.T
    b2 = b.reshape(1, FEATS)

    outs = []
    for n0, n1 in ((0, 2000), (2000, 6000), (6000, N_PTS)):
        n_pts = n1 - n0
        nchunk = n_pts * K_NBR // (NW * CHUNK)
        idx3 = idx_t[:, n0:n1].reshape(NW, nchunk, CHUNK)
        g3 = _sc_gather(x2d, idx3, n_pts).reshape(K_NBR, n_pts, FEATS)
        outs.append(_tc_compute(g3, v[n0:n1], awr, awb, wt, b2, n0))
    return jnp.concatenate(outs, axis=0).reshape(1, N_PTS, FEATS)


# speedup vs baseline: 1.9166x; 1.0538x over previous
import functools

import jax
import jax.numpy as jnp
from jax import lax
from jax.experimental import pallas as pl
from jax.experimental.pallas import tpu as pltpu
from jax.experimental.pallas import tpu_sc as plsc

N_PTS = 10000
K_NBR = 16
FEATS = 128
S_DIM = 8

NW = 32
CHUNK = 40
NB = 400
P_SPLIT = 4000


def _sc_gather(x2d, idx3, n_pts):
    rows = n_pts * K_NBR
    b_per_w = rows // NW
    nchunk = b_per_w // CHUNK
    m = nchunk // 4
    r = nchunk - 4 * m
    mesh = plsc.VectorSubcoreMesh(core_axis_name="c", subcore_axis_name="s")

    @functools.partial(
        pl.kernel,
        out_type=jax.ShapeDtypeStruct((rows, FEATS), jnp.float32),
        mesh=mesh,
        scratch_types=[
            pltpu.VMEM((nchunk, CHUNK), jnp.int32),
            pltpu.VMEM((CHUNK, FEATS), jnp.float32),
            pltpu.VMEM((CHUNK, FEATS), jnp.float32),
            pltpu.VMEM((CHUNK, FEATS), jnp.float32),
            pltpu.VMEM((CHUNK, FEATS), jnp.float32),
            pltpu.SemaphoreType.DMA,
            pltpu.SemaphoreType.DMA,
            pltpu.SemaphoreType.DMA,
            pltpu.SemaphoreType.DMA,
        ],
    )
    def k(x_hbm, idx_hbm, out_hbm, idx_v, b0, b1, b2, b3, s0, s1, s2, s3):
        wid = lax.axis_index("s") * 2 + lax.axis_index("c")
        base = wid * b_per_w
        bufs = (b0, b1, b2, b3)
        sems = (s0, s1, s2, s3)
        pltpu.sync_copy(idx_hbm.at[wid], idx_v)

        def gather(c, l):
            pltpu.make_async_copy(x_hbm.at[idx_v.at[c]], bufs[l], sems[l]).start()

        def gather_wait(l):
            pltpu.make_async_copy(x_hbm.at[pl.ds(0, CHUNK)], bufs[l], sems[l]).wait()

        def write(c, l):
            pltpu.make_async_copy(bufs[l], out_hbm.at[pl.ds(base + c * CHUNK, CHUNK)],
                                  sems[l]).start()

        def write_wait(l):
            pltpu.make_async_copy(bufs[l], out_hbm.at[pl.ds(base, CHUNK)],
                                  sems[l]).wait()

        for l in range(4):
            gather(l, l)

        def step(j, carry):
            for l in range(4):
                gather_wait(l)
                write(4 * j + l, l)
            for l in range(4):
                write_wait(l)
                gather(4 * j + 4 + l, l)
            return carry

        lax.fori_loop(0, m - 1, step, 0)
        for l in range(4):
            gather_wait(l)
            write(4 * (m - 1) + l, l)
        for i in range(r):
            write_wait(i)
            gather(4 * m + i, i)
        for i in range(r):
            gather_wait(i)
            write(4 * m + i, i)
        for l in range(4):
            write_wait(l)

    return k(x2d, idx3)


def _tc_body(g_ref, v_ref, awr_ref, awb_ref, wt_ref, b_ref, carry_ref, o_ref, *, n_off):
    del carry_ref
    i = pl.program_id(0)
    vs = v_ref[...]
    adjw = jnp.dot(vs, awr_ref[...], preferred_element_type=jnp.float32)
    acc = None
    for t in range(K_NBR):
        xt = None
        for k in range(K_NBR):
            c = t * K_NBR + k
            gk = g_ref[k]
            if k % 2 == 0:
                wb = jnp.dot(vs, awb_ref[:, c * FEATS:(c + 1) * FEATS],
                             preferred_element_type=jnp.float32)
                term = wb * gk
            else:
                term = adjw[:, c:c + 1] * gk
            xt = term if xt is None else xt + term
        xt = jnp.where(xt > 0, xt, jnp.exp(xt) - 1.0)
        p = jnp.dot(xt, wt_ref[t * FEATS:(t + 1) * FEATS, :],
                    preferred_element_type=jnp.float32)
        acc = p if acc is None else acc + p
    y = acc + b_ref[...]
    y = jnp.where(y > 0, y, jnp.exp(y) - 1.0)
    rows = n_off + i * NB + lax.broadcasted_iota(jnp.int32, (NB, FEATS), 0)
    o_ref[...] = jnp.where(rows == N_PTS - 1, 0.0, y)


def _tc_compute(g3, v, awr, awb, wt, b2, n_off, carry, interpret=False):
    n_pts = g3.shape[1]
    off = n_off // NB
    return pl.pallas_call(
        functools.partial(_tc_body, n_off=n_off),
        grid=(n_pts // NB,),
        in_specs=[
            pl.BlockSpec((K_NBR, NB, FEATS), lambda i: (0, i, 0)),
            pl.BlockSpec((NB, S_DIM), lambda i, off=off: (i + off, 0)),
            pl.BlockSpec((S_DIM, K_NBR * K_NBR), lambda i: (0, 0)),
            pl.BlockSpec((S_DIM, K_NBR * K_NBR * FEATS), lambda i: (0, 0)),
            pl.BlockSpec((K_NBR * FEATS, FEATS), lambda i: (0, 0)),
            pl.BlockSpec((1, FEATS), lambda i: (0, 0)),
            pl.BlockSpec(memory_space=pl.ANY),
        ],
        out_specs=pl.BlockSpec((NB, FEATS), lambda i, off=off: (i + off, 0)),
        out_shape=jax.ShapeDtypeStruct((N_PTS, FEATS), jnp.float32),
        input_output_aliases={6: 0},
        interpret=interpret,
    )(g3, v, awr, awb, wt, b2, carry)


def kernel(x, t_vertex, neighbor_index, v, adjweight, W, b):
    x2d = x.reshape(N_PTS, FEATS)
    idx_t = neighbor_index.reshape(N_PTS, K_NBR).T
    awr = adjweight.transpose(0, 2, 1).reshape(S_DIM, K_NBR * K_NBR)
    awb = jnp.broadcast_to(awr[:, :, None],
                           (S_DIM, K_NBR * K_NBR, FEATS)).reshape(S_DIM, -1)
    wt = W.T
    b2 = b.reshape(1, FEATS)

    out = jnp.zeros((N_PTS, FEATS), jnp.float32)
    for n0, n1 in ((0, 2000), (2000, 6000), (6000, N_PTS)):
        n_pts = n1 - n0
        nchunk = n_pts * K_NBR // (NW * CHUNK)
        idx3 = idx_t[:, n0:n1].reshape(NW, nchunk, CHUNK)
        g3 = _sc_gather(x2d, idx3, n_pts).reshape(K_NBR, n_pts, FEATS)
        out = _tc_compute(g3, v, awr, awb, wt, b2, n0, out)
    return out.reshape(1, N_PTS, FEATS)


# speedup vs baseline: 1.9260x; 1.0049x over previous
import functools

import jax
import jax.numpy as jnp
from jax import lax
from jax.experimental import pallas as pl
from jax.experimental.pallas import tpu as pltpu
from jax.experimental.pallas import tpu_sc as plsc

N_PTS = 10000
K_NBR = 16
FEATS = 128
S_DIM = 8

NW = 32
CHUNK = 40
NB = 400
P_SPLIT = 4000


def _sc_gather(x2d, idx3, n_pts):
    rows = n_pts * K_NBR
    b_per_w = rows // NW
    nchunk = b_per_w // CHUNK
    m = nchunk // 4
    r = nchunk - 4 * m
    mesh = plsc.VectorSubcoreMesh(core_axis_name="c", subcore_axis_name="s")

    @functools.partial(
        pl.kernel,
        out_type=jax.ShapeDtypeStruct((rows, FEATS), jnp.float32),
        mesh=mesh,
        scratch_types=[
            pltpu.VMEM((nchunk, CHUNK), jnp.int32),
            pltpu.VMEM((CHUNK, FEATS), jnp.float32),
            pltpu.VMEM((CHUNK, FEATS), jnp.float32),
            pltpu.VMEM((CHUNK, FEATS), jnp.float32),
            pltpu.VMEM((CHUNK, FEATS), jnp.float32),
            pltpu.SemaphoreType.DMA,
            pltpu.SemaphoreType.DMA,
            pltpu.SemaphoreType.DMA,
            pltpu.SemaphoreType.DMA,
        ],
    )
    def k(x_hbm, idx_hbm, out_hbm, idx_v, b0, b1, b2, b3, s0, s1, s2, s3):
        wid = lax.axis_index("s") * 2 + lax.axis_index("c")
        base = wid * b_per_w
        bufs = (b0, b1, b2, b3)
        sems = (s0, s1, s2, s3)
        pltpu.sync_copy(idx_hbm.at[wid], idx_v)

        def gather(c, l):
            pltpu.make_async_copy(x_hbm.at[idx_v.at[c]], bufs[l], sems[l]).start()

        def gather_wait(l):
            pltpu.make_async_copy(x_hbm.at[pl.ds(0, CHUNK)], bufs[l], sems[l]).wait()

        def write(c, l):
            pltpu.make_async_copy(bufs[l], out_hbm.at[pl.ds(base + c * CHUNK, CHUNK)],
                                  sems[l]).start()

        def write_wait(l):
            pltpu.make_async_copy(bufs[l], out_hbm.at[pl.ds(base, CHUNK)],
                                  sems[l]).wait()

        for l in range(4):
            gather(l, l)

        def step(j, carry):
            for l in range(4):
                gather_wait(l)
                write(4 * j + l, l)
            for l in range(4):
                write_wait(l)
                gather(4 * j + 4 + l, l)
            return carry

        lax.fori_loop(0, m - 1, step, 0)
        for l in range(4):
            gather_wait(l)
            write(4 * (m - 1) + l, l)
        for i in range(r):
            write_wait(i)
            gather(4 * m + i, i)
        for i in range(r):
            gather_wait(i)
            write(4 * m + i, i)
        for l in range(4):
            write_wait(l)

    return k(x2d, idx3)


def _tc_body(g_ref, v_ref, awr_ref, awb_ref, wt_ref, b_ref, *rest, n_off, has_carry):
    o_ref = rest[-1]
    i = pl.program_id(0)
    vs = v_ref[...]
    adjw = jnp.dot(vs, awr_ref[...], preferred_element_type=jnp.float32)
    acc = None
    for t in range(K_NBR):
        xt = None
        for k in range(K_NBR):
            c = t * K_NBR + k
            gk = g_ref[k]
            if k % 2 == 0:
                wb = jnp.dot(vs, awb_ref[:, c * FEATS:(c + 1) * FEATS],
                             preferred_element_type=jnp.float32)
                term = wb * gk
            else:
                term = adjw[:, c:c + 1] * gk
            xt = term if xt is None else xt + term
        xt = jnp.where(xt > 0, xt, jnp.exp(xt) - 1.0)
        p = jnp.dot(xt, wt_ref[t * FEATS:(t + 1) * FEATS, :],
                    preferred_element_type=jnp.float32)
        acc = p if acc is None else acc + p
    y = acc + b_ref[...]
    y = jnp.where(y > 0, y, jnp.exp(y) - 1.0)
    rows = n_off + i * NB + lax.broadcasted_iota(jnp.int32, (NB, FEATS), 0)
    o_ref[...] = jnp.where(rows == N_PTS - 1, 0.0, y)


def _tc_compute(g3, v, awr, awb, wt, b2, n_off, carry, interpret=False):
    n_pts = g3.shape[1]
    off = n_off // NB
    body = functools.partial(_tc_body, n_off=n_off, has_carry=carry is not None)
    in_specs = [
        pl.BlockSpec((K_NBR, NB, FEATS), lambda i: (0, i, 0)),
        pl.BlockSpec((NB, S_DIM), lambda i, off=off: (i + off, 0)),
        pl.BlockSpec((S_DIM, K_NBR * K_NBR), lambda i: (0, 0)),
        pl.BlockSpec((S_DIM, K_NBR * K_NBR * FEATS), lambda i: (0, 0)),
        pl.BlockSpec((K_NBR * FEATS, FEATS), lambda i: (0, 0)),
        pl.BlockSpec((1, FEATS), lambda i: (0, 0)),
    ]
    args = [g3, v, awr, awb, wt, b2]
    aliases = {}
    if carry is not None:
        in_specs.append(pl.BlockSpec(memory_space=pl.ANY))
        args.append(carry)
        aliases = {6: 0}
    return pl.pallas_call(
        body,
        grid=(n_pts // NB,),
        in_specs=in_specs,
        out_specs=pl.BlockSpec((NB, FEATS), lambda i, off=off: (i + off, 0)),
        out_shape=jax.ShapeDtypeStruct((N_PTS, FEATS), jnp.float32),
        input_output_aliases=aliases,
        interpret=interpret,
    )(*args)


def kernel(x, t_vertex, neighbor_index, v, adjweight, W, b):
    x2d = x.reshape(N_PTS, FEATS)
    idx_t = neighbor_index.reshape(N_PTS, K_NBR).T
    awr = adjweight.transpose(0, 2, 1).reshape(S_DIM, K_NBR * K_NBR)
    awb = jnp.broadcast_to(awr[:, :, None],
                           (S_DIM, K_NBR * K_NBR, FEATS)).reshape(S_DIM, -1)
    wt = W.T
    b2 = b.reshape(1, FEATS)

    out = None
    for n0, n1 in ((0, 2000), (2000, 6000), (6000, N_PTS)):
        n_pts = n1 - n0
        nchunk = n_pts * K_NBR // (NW * CHUNK)
        idx3 = idx_t[:, n0:n1].reshape(NW, nchunk, CHUNK)
        g3 = _sc_gather(x2d, idx3, n_pts).reshape(K_NBR, n_pts, FEATS)
        out = _tc_compute(g3, v, awr, awb, wt, b2, n0, out)
    return out.reshape(1, N_PTS, FEATS)


# speedup vs baseline: 1.9373x; 1.0058x over previous
import functools

import jax
import jax.numpy as jnp
from jax import lax
from jax.experimental import pallas as pl
from jax.experimental.pallas import tpu as pltpu
from jax.experimental.pallas import tpu_sc as plsc

N_PTS = 10000
K_NBR = 16
FEATS = 128
S_DIM = 8

NW = 32
CHUNK = 40
NB = 400
P_SPLIT = 4000


def _sc_gather(x2d, idx3, n_pts):
    rows = n_pts * K_NBR
    b_per_w = rows // NW
    nchunk = b_per_w // CHUNK
    m = nchunk // 4
    r = nchunk - 4 * m
    mesh = plsc.VectorSubcoreMesh(core_axis_name="c", subcore_axis_name="s")

    @functools.partial(
        pl.kernel,
        out_type=jax.ShapeDtypeStruct((rows, FEATS), jnp.float32),
        mesh=mesh,
        scratch_types=[
            pltpu.VMEM((nchunk, CHUNK), jnp.int32),
            pltpu.VMEM((CHUNK, FEATS), jnp.float32),
            pltpu.VMEM((CHUNK, FEATS), jnp.float32),
            pltpu.VMEM((CHUNK, FEATS), jnp.float32),
            pltpu.VMEM((CHUNK, FEATS), jnp.float32),
            pltpu.SemaphoreType.DMA,
            pltpu.SemaphoreType.DMA,
            pltpu.SemaphoreType.DMA,
            pltpu.SemaphoreType.DMA,
        ],
    )
    def k(x_hbm, idx_hbm, out_hbm, idx_v, b0, b1, b2, b3, s0, s1, s2, s3):
        wid = lax.axis_index("s") * 2 + lax.axis_index("c")
        base = wid * b_per_w
        bufs = (b0, b1, b2, b3)
        sems = (s0, s1, s2, s3)
        pltpu.sync_copy(idx_hbm.at[wid], idx_v)

        def gather(c, l):
            pltpu.make_async_copy(x_hbm.at[idx_v.at[c]], bufs[l], sems[l]).start()

        def gather_wait(l):
            pltpu.make_async_copy(x_hbm.at[pl.ds(0, CHUNK)], bufs[l], sems[l]).wait()

        def write(c, l):
            pltpu.make_async_copy(bufs[l], out_hbm.at[pl.ds(base + c * CHUNK, CHUNK)],
                                  sems[l]).start()

        def write_wait(l):
            pltpu.make_async_copy(bufs[l], out_hbm.at[pl.ds(base, CHUNK)],
                                  sems[l]).wait()

        for l in range(4):
            gather(l, l)

        def step(j, carry):
            for l in range(4):
                gather_wait(l)
                write(4 * j + l, l)
            for l in range(4):
                write_wait(l)
                gather(4 * j + 4 + l, l)
            return carry

        lax.fori_loop(0, m - 1, step, 0)
        for l in range(4):
            gather_wait(l)
            write(4 * (m - 1) + l, l)
        for i in range(r):
            write_wait(i)
            gather(4 * m + i, i)
        for i in range(r):
            gather_wait(i)
            write(4 * m + i, i)
        for l in range(4):
            write_wait(l)

    return k(x2d, idx3)


def _tc_body(g_ref, v_ref, awr_ref, awb_ref, wt_ref, b_ref, *rest, n_off, has_carry):
    o_ref = rest[-1]
    i = pl.program_id(0)
    vs = v_ref[...]
    adjw = jnp.dot(vs, awr_ref[...], preferred_element_type=jnp.float32)
    acc = None
    for t in range(K_NBR):
        xt = None
        for k in range(K_NBR):
            c = t * K_NBR + k
            gk = g_ref[k]
            if k % 2 == 0:
                wb = jnp.dot(vs, awb_ref[:, c * FEATS:(c + 1) * FEATS],
                             preferred_element_type=jnp.float32)
                term = wb * gk
            else:
                term = adjw[:, c:c + 1] * gk
            xt = term if xt is None else xt + term
        xt = jnp.where(xt > 0, xt, jnp.exp(xt) - 1.0)
        p = jnp.dot(xt, wt_ref[t * FEATS:(t + 1) * FEATS, :],
                    preferred_element_type=jnp.float32)
        acc = p if acc is None else acc + p
    y = acc + b_ref[...]
    y = jnp.where(y > 0, y, jnp.exp(y) - 1.0)
    rows = n_off + i * NB + lax.broadcasted_iota(jnp.int32, (NB, FEATS), 0)
    o_ref[...] = jnp.where(rows == N_PTS - 1, 0.0, y)


def _tc_compute(g3, v, awr, awb, wt, b2, n_off, carry, interpret=False):
    n_pts = g3.shape[1]
    off = n_off // NB
    body = functools.partial(_tc_body, n_off=n_off, has_carry=carry is not None)
    in_specs = [
        pl.BlockSpec((K_NBR, NB, FEATS), lambda i: (0, i, 0)),
        pl.BlockSpec((NB, S_DIM), lambda i, off=off: (i + off, 0)),
        pl.BlockSpec((S_DIM, K_NBR * K_NBR), lambda i: (0, 0)),
        pl.BlockSpec((S_DIM, K_NBR * K_NBR * FEATS), lambda i: (0, 0)),
        pl.BlockSpec((K_NBR * FEATS, FEATS), lambda i: (0, 0)),
        pl.BlockSpec((1, FEATS), lambda i: (0, 0)),
    ]
    args = [g3, v, awr, awb, wt, b2]
    aliases = {}
    if carry is not None:
        in_specs.append(pl.BlockSpec(memory_space=pl.ANY))
        args.append(carry)
        aliases = {6: 0}
    return pl.pallas_call(
        body,
        grid=(n_pts // NB,),
        in_specs=in_specs,
        out_specs=pl.BlockSpec((NB, FEATS), lambda i, off=off: (i + off, 0)),
        out_shape=jax.ShapeDtypeStruct((N_PTS, FEATS), jnp.float32),
        input_output_aliases=aliases,
        interpret=interpret,
    )(*args)


def kernel(x, t_vertex, neighbor_index, v, adjweight, W, b):
    x2d = x.reshape(N_PTS, FEATS)
    idx_t = neighbor_index.reshape(N_PTS, K_NBR).T
    awr = adjweight.transpose(0, 2, 1).reshape(S_DIM, K_NBR * K_NBR)
    awb = jnp.broadcast_to(awr[:, :, None],
                           (S_DIM, K_NBR * K_NBR, FEATS)).reshape(S_DIM, -1)
    wt = W.T
    b2 = b.reshape(1, FEATS)

    out = None
    for n0, n1 in ((0, 400), (400, 2000), (2000, 6000), (6000, N_PTS)):
        n_pts = n1 - n0
        nchunk = n_pts * K_NBR // (NW * CHUNK)
        idx3 = idx_t[:, n0:n1].reshape(NW, nchunk, CHUNK)
        g3 = _sc_gather(x2d, idx3, n_pts).reshape(K_NBR, n_pts, FEATS)
        out = _tc_compute(g3, v, awr, awb, wt, b2, n0, out)
    return out.reshape(1, N_PTS, FEATS)


# speedup vs baseline: 1.9949x; 1.0297x over previous
import functools

import jax
import jax.numpy as jnp
from jax import lax
from jax.experimental import pallas as pl
from jax.experimental.pallas import tpu as pltpu
from jax.experimental.pallas import tpu_sc as plsc

N_PTS = 10000
K_NBR = 16
FEATS = 128
S_DIM = 8

NW = 32
CHUNK = 40
NB = 400
P_SPLIT = 4000


def _sc_gather(x2d, idx3, n_pts):
    rows = n_pts * K_NBR
    b_per_w = rows // NW
    nchunk = b_per_w // CHUNK
    m = nchunk // 4
    r = nchunk - 4 * m
    mesh = plsc.VectorSubcoreMesh(core_axis_name="c", subcore_axis_name="s")

    @functools.partial(
        pl.kernel,
        out_type=jax.ShapeDtypeStruct((rows, FEATS), jnp.float32),
        mesh=mesh,
        scratch_types=[
            pltpu.VMEM((nchunk, CHUNK), jnp.int32),
            pltpu.VMEM((CHUNK, FEATS), jnp.float32),
            pltpu.VMEM((CHUNK, FEATS), jnp.float32),
            pltpu.VMEM((CHUNK, FEATS), jnp.float32),
            pltpu.VMEM((CHUNK, FEATS), jnp.float32),
            pltpu.SemaphoreType.DMA,
            pltpu.SemaphoreType.DMA,
            pltpu.SemaphoreType.DMA,
            pltpu.SemaphoreType.DMA,
        ],
    )
    def k(x_hbm, idx_hbm, out_hbm, idx_v, b0, b1, b2, b3, s0, s1, s2, s3):
        wid = lax.axis_index("s") * 2 + lax.axis_index("c")
        base = wid * b_per_w
        bufs = (b0, b1, b2, b3)
        sems = (s0, s1, s2, s3)
        pltpu.sync_copy(idx_hbm.at[wid], idx_v)

        def gather(c, l):
            pltpu.make_async_copy(x_hbm.at[idx_v.at[c]], bufs[l], sems[l]).start()

        def gather_wait(l):
            pltpu.make_async_copy(x_hbm.at[pl.ds(0, CHUNK)], bufs[l], sems[l]).wait()

        def write(c, l):
            pltpu.make_async_copy(bufs[l], out_hbm.at[pl.ds(base + c * CHUNK, CHUNK)],
                                  sems[l]).start()

        def write_wait(l):
            pltpu.make_async_copy(bufs[l], out_hbm.at[pl.ds(base, CHUNK)],
                                  sems[l]).wait()

        for l in range(4):
            gather(l, l)

        def step(j, carry):
            for l in range(4):
                gather_wait(l)
                write(4 * j + l, l)
            for l in range(4):
                write_wait(l)
                gather(4 * j + 4 + l, l)
            return carry

        lax.fori_loop(0, m - 1, step, 0)
        for l in range(4):
            gather_wait(l)
            write(4 * (m - 1) + l, l)
        for i in range(r):
            write_wait(i)
            gather(4 * m + i, i)
        for i in range(r):
            gather_wait(i)
            write(4 * m + i, i)
        for l in range(4):
            write_wait(l)

    return k(x2d, idx3)


def _tc_body(g_ref, v_ref, awr_ref, awb_ref, wt_ref, b_ref, *rest, n_off, has_carry):
    o_ref = rest[-1]
    i = pl.program_id(0)
    vs = v_ref[...]
    adjw = jnp.dot(vs, awr_ref[...], preferred_element_type=jnp.float32)
    acc = None
    for t in range(K_NBR):
        xt = None
        for k in range(K_NBR):
            c = t * K_NBR + k
            gk = g_ref[k]
            if k % 4 != 0:
                wb = jnp.dot(vs, awb_ref[:, c * FEATS:(c + 1) * FEATS],
                             preferred_element_type=jnp.float32)
                term = wb * gk
            else:
                term = adjw[:, c:c + 1] * gk
            xt = term if xt is None else xt + term
        xt = jnp.where(xt > 0, xt, jnp.exp(xt) - 1.0)
        p = jnp.dot(xt, wt_ref[t * FEATS:(t + 1) * FEATS, :],
                    preferred_element_type=jnp.float32)
        acc = p if acc is None else acc + p
    y = acc + b_ref[...]
    y = jnp.where(y > 0, y, jnp.exp(y) - 1.0)
    rows = n_off + i * NB + lax.broadcasted_iota(jnp.int32, (NB, FEATS), 0)
    o_ref[...] = jnp.where(rows == N_PTS - 1, 0.0, y)


def _tc_compute(g3, v, awr, awb, wt, b2, n_off, carry, interpret=False):
    n_pts = g3.shape[1]
    off = n_off // NB
    body = functools.partial(_tc_body, n_off=n_off, has_carry=carry is not None)
    in_specs = [
        pl.BlockSpec((K_NBR, NB, FEATS), lambda i: (0, i, 0)),
        pl.BlockSpec((NB, S_DIM), lambda i, off=off: (i + off, 0)),
        pl.BlockSpec((S_DIM, K_NBR * K_NBR), lambda i: (0, 0)),
        pl.BlockSpec((S_DIM, K_NBR * K_NBR * FEATS), lambda i: (0, 0)),
        pl.BlockSpec((K_NBR * FEATS, FEATS), lambda i: (0, 0)),
        pl.BlockSpec((1, FEATS), lambda i: (0, 0)),
    ]
    args = [g3, v, awr, awb, wt, b2]
    aliases = {}
    if carry is not None:
        in_specs.append(pl.BlockSpec(memory_space=pl.ANY))
        args.append(carry)
        aliases = {6: 0}
    return pl.pallas_call(
        body,
        grid=(n_pts // NB,),
        in_specs=in_specs,
        out_specs=pl.BlockSpec((NB, FEATS), lambda i, off=off: (i + off, 0)),
        out_shape=jax.ShapeDtypeStruct((N_PTS, FEATS), jnp.float32),
        input_output_aliases=aliases,
        interpret=interpret,
    )(*args)


def kernel(x, t_vertex, neighbor_index, v, adjweight, W, b):
    x2d = x.reshape(N_PTS, FEATS)
    idx_t = neighbor_index.reshape(N_PTS, K_NBR).T
    awr = adjweight.transpose(0, 2, 1).reshape(S_DIM, K_NBR * K_NBR)
    awb = jnp.broadcast_to(awr[:, :, None],
                           (S_DIM, K_NBR * K_NBR, FEATS)).reshape(S_DIM, -1)
    wt = W.T
    b2 = b.reshape(1, FEATS)

    out = None
    for n0, n1 in ((0, 400), (400, 2000), (2000, 6000), (6000, N_PTS)):
        n_pts = n1 - n0
        nchunk = n_pts * K_NBR // (NW * CHUNK)
        idx3 = idx_t[:, n0:n1].reshape(NW, nchunk, CHUNK)
        g3 = _sc_gather(x2d, idx3, n_pts).reshape(K_NBR, n_pts, FEATS)
        out = _tc_compute(g3, v, awr, awb, wt, b2, n0, out)
    return out.reshape(1, N_PTS, FEATS)
